# baseline (speedup 1.0000x reference)
import functools

import jax
import jax.numpy as jnp
from jax import lax
from jax.experimental import pallas as pl
from jax.experimental.pallas import tpu as pltpu
from jax.experimental.pallas import tpu_sc as plsc

_V = 1024
_K = 4
_E = 32768
_NC, _NS = 2, 16
_NW = _NC * _NS
_EPW = _E // _NW
_SLICE = _V * _V // _NS


def _scm_body(e_ref, out_ref, src_v, dst_v, idx_v, ones_v, zb_v, m_sh):
    cid = lax.axis_index("c")
    sid = lax.axis_index("s")
    wid = sid * _NC + cid

    def _zf(i, c):
        zb_v[pl.ds(i * 16, 16)] = jnp.zeros((16,), jnp.float32)
        return c

    lax.fori_loop(0, zb_v.shape[0] // 16, _zf, 0)
    for j in range(_SLICE // zb_v.shape[0]):
        pltpu.sync_copy(
            zb_v, m_sh.at[pl.ds(sid * _SLICE + j * zb_v.shape[0],
                                zb_v.shape[0])])
    for j in range(ones_v.shape[0] // 16):
        ones_v[pl.ds(j * 16, 16)] = jnp.ones((16,), jnp.float32)
    plsc.subcore_barrier()

    base = wid * _EPW
    pltpu.sync_copy(e_ref.at[0, pl.ds(base, _EPW)], src_v)
    pltpu.sync_copy(e_ref.at[1, pl.ds(base, _EPW)], dst_v)
    for j in range(_EPW // 128):
        def _bf(i, c, j=j):
            o = j * 128 + i * 16
            s16 = src_v[pl.ds(o, 16)]
            d16 = dst_v[pl.ds(o, 16)]
            idx_v[j, pl.ds(i * 16, 16)] = s16 * _V + d16
            return c

        lax.fori_loop(0, 8, _bf, 0)
    for j in range(_EPW // 128):
        pltpu.sync_copy(ones_v, m_sh.at[idx_v.at[j]], add=True)
    plsc.subcore_barrier()
    pltpu.sync_copy(m_sh.at[pl.ds(sid * _SLICE, _SLICE)],
                    out_ref.at[cid, pl.ds(sid * _SLICE, _SLICE)])


_scm_kernel = functools.partial(
    pl.kernel,
    out_type=jax.ShapeDtypeStruct((_NC, _V * _V), jnp.float32),
    mesh=plsc.VectorSubcoreMesh(core_axis_name="c", subcore_axis_name="s"),
    scratch_types=[
        pltpu.VMEM((_EPW,), jnp.int32),
        pltpu.VMEM((_EPW,), jnp.int32),
        pltpu.VMEM((_EPW // 128, 128), jnp.int32),
        pltpu.VMEM((128,), jnp.float32),
        pltpu.VMEM((8192,), jnp.float32),
        pltpu.VMEM_SHARED((_V * _V,), jnp.float32),
    ],
)(_scm_body)


def _build_mt(edge_index):
    return _scm_kernel(edge_index)


def _norm_body(m_ref, hi_ref, lo_ref):
    mt = (m_ref[0] + m_ref[1]).reshape(_V, _V)
    dout = jnp.sum(mt, axis=1, keepdims=True)
    din = jnp.sum(mt, axis=0, keepdims=True)
    amat = -(lax.rsqrt(jnp.maximum(dout, 1.0)) * mt
             * lax.rsqrt(jnp.maximum(din, 1.0)))
    hp = lax.Precision.HIGHEST
    a2 = jnp.dot(amat, amat, preferred_element_type=jnp.float32, precision=hp)
    eye = (lax.broadcasted_iota(jnp.int32, (_V, _V), 0)
           == lax.broadcasted_iota(jnp.int32, (_V, _V), 1)).astype(jnp.float32)
    p2 = 2.0 * a2 - eye
    p3 = 2.0 * jnp.dot(amat, p2, preferred_element_type=jnp.float32,
                       precision=hp) - amat
    pcat = jnp.concatenate([amat, p2, p3], axis=1)
    hi = pcat.astype(jnp.bfloat16)
    hi_ref[...] = hi
    lo_ref[...] = (pcat - hi.astype(jnp.float32)).astype(jnp.bfloat16)


def _normalize(mt):
    return pl.pallas_call(
        _norm_body,
        out_shape=[
            jax.ShapeDtypeStruct((_V, 3 * _V), jnp.bfloat16),
            jax.ShapeDtypeStruct((_V, 3 * _V), jnp.bfloat16),
        ],
    )(mt)


def _layer_body(ac_ref, h_ref, hi_ref, lo_ref, w_ref, y_ref, st_ref, *,
                first, nb, fin):
    b = pl.program_id(0)
    x0 = h_ref[...]
    if not first:
        ac = ac_ref[...]
        x0 = jnp.maximum(ac[None, :, 0:1] * x0 + ac[None, :, 1:2], 0.0)
    x0s = x0.reshape(nb * fin, _V)
    p_hi = hi_ref[...]
    p_lo = lo_ref[...]
    x_hi = x0s.astype(jnp.bfloat16)
    x_lo = (x0s - x_hi.astype(jnp.float32)).astype(jnp.bfloat16)
    xk = jnp.dot(x_hi, p_lo, preferred_element_type=jnp.float32)
    xk += jnp.dot(x_lo, p_hi, preferred_element_type=jnp.float32)
    xk += jnp.dot(x_hi, p_hi, preferred_element_type=jnp.float32)
    w = w_ref[...]
    wk = [w[:, k * fin:(k + 1) * fin] for k in range(_K)]
    sq = jnp.zeros_like(st_ref)
    for i in range(nb):
        r0, r1 = i * fin, (i + 1) * fin
        y = jnp.dot(wk[0], x0s[r0:r1], preferred_element_type=jnp.float32)
        for k in range(1, _K):
            y += jnp.dot(wk[k], xk[r0:r1, (k - 1) * _V:k * _V],
                         preferred_element_type=jnp.float32)
        y_ref[i] = y
        s = jnp.sum(y, axis=1, keepdims=True)
        q = jnp.sum(y * y, axis=1, keepdims=True)
        sq += jnp.concatenate([s, q], axis=1)

    @pl.when(b == 0)
    def _():
        st_ref[...] = jnp.zeros_like(st_ref)

    st_ref[...] += sq


def _layer(h, a_hi, a_lo, wr, ac, first, nb):
    bsz, fin, _ = h.shape
    fout = wr.shape[0]
    return pl.pallas_call(
        functools.partial(_layer_body, first=first, nb=nb, fin=fin),
        grid=(bsz // nb,),
        in_specs=[
            pl.BlockSpec((fin, 2), lambda b: (0, 0)),
            pl.BlockSpec((nb, fin, _V), lambda b: (b, 0, 0)),
            pl.BlockSpec((_V, 3 * _V), lambda b: (0, 0)),
            pl.BlockSpec((_V, 3 * _V), lambda b: (0, 0)),
            pl.BlockSpec((fout, _K * fin), lambda b: (0, 0)),
        ],
        out_specs=[
            pl.BlockSpec((nb, fout, _V), lambda b: (b, 0, 0)),
            pl.BlockSpec((fout, 2), lambda b: (0, 0)),
        ],
        out_shape=[
            jax.ShapeDtypeStruct((bsz, fout, _V), jnp.float32),
            jax.ShapeDtypeStruct((fout, 2), jnp.float32),
        ],
    )(ac, h, a_hi, a_lo, wr)


_CC = 8


def _cls_body(h_ref, ac_ref, w_ref, bc_ref, o_ref):
    c = pl.program_id(0)

    @pl.when(c == 0)
    def _():
        o_ref[...] = jnp.broadcast_to(bc_ref[...], o_ref.shape)

    acc = jnp.zeros_like(o_ref)
    for j in range(_CC):
        hc = h_ref[:, j, :]
        hn = jnp.maximum(ac_ref[j, 0] * hc + ac_ref[j, 1], 0.0)
        acc += jnp.dot(hn, w_ref[j], preferred_element_type=jnp.float32)
    o_ref[...] += acc


def _classifier(h, ac, wc7, bc):
    bsz, nch, _ = h.shape
    ncls = wc7.shape[2]
    return pl.pallas_call(
        _cls_body,
        grid=(nch // _CC,),
        in_specs=[
            pl.BlockSpec((bsz, _CC, _V), lambda c: (0, c, 0)),
            pl.BlockSpec((_CC, 2), lambda c: (c, 0)),
            pl.BlockSpec((_CC, _V, ncls), lambda c: (c, 0, 0)),
            pl.BlockSpec((1, ncls), lambda c: (0, 0)),
        ],
        out_specs=pl.BlockSpec((bsz, ncls), lambda c: (0, 0)),
        out_shape=jax.ShapeDtypeStruct((bsz, ncls), jnp.float32),
    )(h, ac, wc7, bc[None, :])


def _fold_bn(st, g, b, n):
    s, q = st[:, 0], st[:, 1]
    m = s / n
    v = q / n - m * m
    scale = g * lax.rsqrt(v + 1e-5)
    shift = b - m * scale
    return jnp.stack([scale, shift], axis=1)


def kernel(x, edge_index, W0, W1, W2, W3, W4, W5, W6, g0, g1, g2, g3, g4, g5, g6, b0, b1, b2, b3, b4, b5, b6, Wc, bc):
    ws = [W0, W1, W2, W3, W4, W5, W6]
    gs = [g0, g1, g2, g3, g4, g5, g6]
    bs = [b0, b1, b2, b3, b4, b5, b6]

    mt = _build_mt(edge_index)
    a_hi, a_lo = _normalize(mt)

    bsz = x.shape[0]
    n = float(bsz * _V)
    h = x
    ac = None
    for li, w in enumerate(ws):
        fin = h.shape[1]
        fout = w.shape[0]
        wr = w.reshape(fout, fin, _K).transpose(0, 2, 1).reshape(fout, _K * fin)
        if li == 0:
            ac = jnp.zeros((fin, 2), jnp.float32)
        nb = 16 if fin <= 8 else (8 if fin <= 96 else 4)
        h, st = _layer(h, a_hi, a_lo, wr, ac, first=(li == 0), nb=nb)
        ac = _fold_bn(st, gs[li], bs[li], n)

    ncls = Wc.shape[0]
    wc7 = Wc.reshape(ncls, h.shape[1], _V).transpose(1, 2, 0)
    return _classifier(h, ac, wc7, bc)


# speedup vs baseline: 1.0268x; 1.0268x over previous
import functools

import jax
import jax.numpy as jnp
from jax import lax
from jax.experimental import pallas as pl
from jax.experimental.pallas import tpu as pltpu
from jax.experimental.pallas import tpu_sc as plsc

_V = 1024
_K = 4
_E = 32768
_NC, _NS = 2, 16
_NW = _NC * _NS
_EPW = _E // _NW
_SLICE = _V * _V // _NS


def _scm_body(e_ref, out_ref, src_v, dst_v, idx_v, ones_v, zb_v, m_sh):
    cid = lax.axis_index("c")
    sid = lax.axis_index("s")
    wid = sid * _NC + cid

    def _zf(i, c):
        zb_v[pl.ds(i * 16, 16)] = jnp.zeros((16,), jnp.float32)
        return c

    lax.fori_loop(0, zb_v.shape[0] // 16, _zf, 0)
    for j in range(_SLICE // zb_v.shape[0]):
        pltpu.sync_copy(
            zb_v, m_sh.at[pl.ds(sid * _SLICE + j * zb_v.shape[0],
                                zb_v.shape[0])])
    for j in range(ones_v.shape[0] // 16):
        ones_v[pl.ds(j * 16, 16)] = jnp.ones((16,), jnp.float32)
    plsc.subcore_barrier()

    base = wid * _EPW
    pltpu.sync_copy(e_ref.at[0, pl.ds(base, _EPW)], src_v)
    pltpu.sync_copy(e_ref.at[1, pl.ds(base, _EPW)], dst_v)
    for j in range(_EPW // 128):
        def _bf(i, c, j=j):
            o = j * 128 + i * 16
            s16 = src_v[pl.ds(o, 16)]
            d16 = dst_v[pl.ds(o, 16)]
            idx_v[j, pl.ds(i * 16, 16)] = s16 * _V + d16
            return c

        lax.fori_loop(0, 8, _bf, 0)
    for j in range(_EPW // 128):
        pltpu.sync_copy(ones_v, m_sh.at[idx_v.at[j]], add=True)
    plsc.subcore_barrier()
    pltpu.sync_copy(m_sh.at[pl.ds(sid * _SLICE, _SLICE)],
                    out_ref.at[cid, pl.ds(sid * _SLICE, _SLICE)])


_scm_kernel = functools.partial(
    pl.kernel,
    out_type=jax.ShapeDtypeStruct((_NC, _V * _V), jnp.float32),
    mesh=plsc.VectorSubcoreMesh(core_axis_name="c", subcore_axis_name="s"),
    scratch_types=[
        pltpu.VMEM((_EPW,), jnp.int32),
        pltpu.VMEM((_EPW,), jnp.int32),
        pltpu.VMEM((_EPW // 128, 128), jnp.int32),
        pltpu.VMEM((128,), jnp.float32),
        pltpu.VMEM((8192,), jnp.float32),
        pltpu.VMEM_SHARED((_V * _V,), jnp.float32),
    ],
)(_scm_body)


def _build_mt(edge_index):
    return _scm_kernel(edge_index)


def _norm_body(m_ref, hi_ref, lo_ref):
    mt = (m_ref[0] + m_ref[1]).reshape(_V, _V)
    dout = jnp.sum(mt, axis=1, keepdims=True)
    din = jnp.sum(mt, axis=0, keepdims=True)
    amat = -(lax.rsqrt(jnp.maximum(dout, 1.0)) * mt
             * lax.rsqrt(jnp.maximum(din, 1.0)))
    hp = lax.Precision.HIGHEST
    a2 = jnp.dot(amat, amat, preferred_element_type=jnp.float32, precision=hp)
    eye = (lax.broadcasted_iota(jnp.int32, (_V, _V), 0)
           == lax.broadcasted_iota(jnp.int32, (_V, _V), 1)).astype(jnp.float32)
    p2 = 2.0 * a2 - eye
    p3 = 2.0 * jnp.dot(amat, p2, preferred_element_type=jnp.float32,
                       precision=hp) - amat
    pcat = jnp.concatenate([amat, p2, p3], axis=1)
    hi = pcat.astype(jnp.bfloat16)
    hi_ref[...] = hi
    lo_ref[...] = (pcat - hi.astype(jnp.float32)).astype(jnp.bfloat16)


def _normalize(mt):
    return pl.pallas_call(
        _norm_body,
        out_shape=[
            jax.ShapeDtypeStruct((_V, 3 * _V), jnp.bfloat16),
            jax.ShapeDtypeStruct((_V, 3 * _V), jnp.bfloat16),
        ],
    )(mt)


def _layer_body(ac_ref, h_ref, hi_ref, lo_ref, w_ref, y_ref, st_ref, *,
                first, nb, fin):
    b = pl.program_id(0)
    x0 = h_ref[...]
    if not first:
        ac = ac_ref[...]
        x0 = jnp.maximum(ac[None, :, 0:1] * x0 + ac[None, :, 1:2], 0.0)
    x0s = x0.reshape(nb * fin, _V)
    p_hi = hi_ref[...]
    p_lo = lo_ref[...]
    x_hi = x0s.astype(jnp.bfloat16)
    x_lo = (x0s - x_hi.astype(jnp.float32)).astype(jnp.bfloat16)
    xk = jnp.dot(x_hi, p_lo, preferred_element_type=jnp.float32)
    xk += jnp.dot(x_lo, p_hi, preferred_element_type=jnp.float32)
    xk += jnp.dot(x_hi, p_hi, preferred_element_type=jnp.float32)
    w = w_ref[...]
    sq = jnp.zeros_like(st_ref)
    for i in range(nb):
        r0, r1 = i * fin, (i + 1) * fin
        xs = jnp.concatenate(
            [x0s[r0:r1], xk[r0:r1, :_V], xk[r0:r1, _V:2 * _V],
             xk[r0:r1, 2 * _V:]], axis=0)
        y = jnp.dot(w, xs, preferred_element_type=jnp.float32)
        y_ref[i] = y
        s = jnp.sum(y, axis=1, keepdims=True)
        q = jnp.sum(y * y, axis=1, keepdims=True)
        sq += jnp.concatenate([s, q], axis=1)

    @pl.when(b == 0)
    def _():
        st_ref[...] = jnp.zeros_like(st_ref)

    st_ref[...] += sq


def _layer(h, a_hi, a_lo, wr, ac, first, nb):
    bsz, fin, _ = h.shape
    fout = wr.shape[0]
    return pl.pallas_call(
        functools.partial(_layer_body, first=first, nb=nb, fin=fin),
        grid=(bsz // nb,),
        in_specs=[
            pl.BlockSpec((fin, 2), lambda b: (0, 0)),
            pl.BlockSpec((nb, fin, _V), lambda b: (b, 0, 0)),
            pl.BlockSpec((_V, 3 * _V), lambda b: (0, 0)),
            pl.BlockSpec((_V, 3 * _V), lambda b: (0, 0)),
            pl.BlockSpec((fout, _K * fin), lambda b: (0, 0)),
        ],
        out_specs=[
            pl.BlockSpec((nb, fout, _V), lambda b: (b, 0, 0)),
            pl.BlockSpec((fout, 2), lambda b: (0, 0)),
        ],
        out_shape=[
            jax.ShapeDtypeStruct((bsz, fout, _V), jnp.float32),
            jax.ShapeDtypeStruct((fout, 2), jnp.float32),
        ],
    )(ac, h, a_hi, a_lo, wr)


_CC = 8


def _cls_body(h_ref, ac_ref, w_ref, bc_ref, o_ref):
    c = pl.program_id(0)

    @pl.when(c == 0)
    def _():
        o_ref[...] = jnp.broadcast_to(bc_ref[...], o_ref.shape)

    acc = jnp.zeros_like(o_ref)
    for j in range(_CC):
        hc = h_ref[:, j, :]
        hn = jnp.maximum(ac_ref[j, 0] * hc + ac_ref[j, 1], 0.0)
        acc += jnp.dot(hn, w_ref[j], preferred_element_type=jnp.float32)
    o_ref[...] += acc


def _classifier(h, ac, wc7, bc):
    bsz, nch, _ = h.shape
    ncls = wc7.shape[2]
    return pl.pallas_call(
        _cls_body,
        grid=(nch // _CC,),
        in_specs=[
            pl.BlockSpec((bsz, _CC, _V), lambda c: (0, c, 0)),
            pl.BlockSpec((_CC, 2), lambda c: (c, 0)),
            pl.BlockSpec((_CC, _V, ncls), lambda c: (c, 0, 0)),
            pl.BlockSpec((1, ncls), lambda c: (0, 0)),
        ],
        out_specs=pl.BlockSpec((bsz, ncls), lambda c: (0, 0)),
        out_shape=jax.ShapeDtypeStruct((bsz, ncls), jnp.float32),
    )(h, ac, wc7, bc[None, :])


def _fold_bn(st, g, b, n):
    s, q = st[:, 0], st[:, 1]
    m = s / n
    v = q / n - m * m
    scale = g * lax.rsqrt(v + 1e-5)
    shift = b - m * scale
    return jnp.stack([scale, shift], axis=1)


def kernel(x, edge_index, W0, W1, W2, W3, W4, W5, W6, g0, g1, g2, g3, g4, g5, g6, b0, b1, b2, b3, b4, b5, b6, Wc, bc):
    ws = [W0, W1, W2, W3, W4, W5, W6]
    gs = [g0, g1, g2, g3, g4, g5, g6]
    bs = [b0, b1, b2, b3, b4, b5, b6]

    mt = _build_mt(edge_index)
    a_hi, a_lo = _normalize(mt)

    bsz = x.shape[0]
    n = float(bsz * _V)
    h = x
    ac = None
    for li, w in enumerate(ws):
        fin = h.shape[1]
        fout = w.shape[0]
        wr = w.reshape(fout, fin, _K).transpose(0, 2, 1).reshape(fout, _K * fin)
        if li == 0:
            ac = jnp.zeros((fin, 2), jnp.float32)
        nb = 16 if fin <= 8 else (8 if fin <= 96 else 4)
        h, st = _layer(h, a_hi, a_lo, wr, ac, first=(li == 0), nb=nb)
        ac = _fold_bn(st, gs[li], bs[li], n)

    ncls = Wc.shape[0]
    wc7 = Wc.reshape(ncls, h.shape[1], _V).transpose(1, 2, 0)
    return _classifier(h, ac, wc7, bc)
